# initial kernel scaffold (unmeasured)
import jax
import jax.numpy as jnp
from jax import lax
from jax.experimental import pallas as pl
from jax.experimental.pallas import tpu as pltpu

N_DEV = 8
B = 2
S_PER = 512
N_OUT = 8192
TILE_S = 256
N_TILES = S_PER // TILE_S
HOPS = N_DEV - 1


def _ring_reduce_scatter(partial):

    def body(p_ref, out_ref, acc_hbm, recv_hbm, va, vb, vc, vf32,
             send_sems, recv_sems, sem_a, sem_b, sem_c):
        my = lax.axis_index("i")
        right = lax.rem(my + 1, N_DEV)

        c0 = lax.rem(my + N_DEV - 1, N_DEV)
        init = pltpu.make_async_copy(p_ref.at[c0], acc_hbm, sem_a)
        init.start()
        init.wait()

        for s in range(HOPS):
            rdma = pltpu.make_async_remote_copy(
                src_ref=acc_hbm,
                dst_ref=recv_hbm.at[s],
                send_sem=send_sems.at[s],
                recv_sem=recv_sems.at[s],
                device_id=(right,),
                device_id_type=pl.DeviceIdType.MESH,
            )
            rdma.start()
            rdma.wait()

            c = lax.rem(my + 2 * N_DEV - 2 - s, N_DEV)
            final = s == HOPS - 1
            for b in range(B):
                for t in range(N_TILES):
                    sl = pl.ds(t * TILE_S, TILE_S)
                    cp_r = pltpu.make_async_copy(recv_hbm.at[s, b, sl], va, sem_a)
                    cp_l = pltpu.make_async_copy(p_ref.at[c, b, sl], vb, sem_b)
                    cp_r.start()
                    cp_l.start()
                    cp_r.wait()
                    cp_l.wait()
                    if final:
                        vf32[...] = (va[...].astype(jnp.float32)
                                     + vb[...].astype(jnp.float32))
                        cp_o = pltpu.make_async_copy(vf32, out_ref.at[b, sl], sem_c)
                    else:
                        vc[...] = va[...] + vb[...]
                        cp_o = pltpu.make_async_copy(vc, acc_hbm.at[b, sl], sem_c)
                    cp_o.start()
                    cp_o.wait()

    hbm = pltpu.MemorySpace.HBM
    return pl.pallas_call(
        body,
        out_shape=jax.ShapeDtypeStruct((B, S_PER, N_OUT), jnp.float32),
        in_specs=[pl.BlockSpec(memory_space=hbm)],
        out_specs=pl.BlockSpec(memory_space=hbm),
        scratch_shapes=[
            hbm((B, S_PER, N_OUT), jnp.bfloat16),
            hbm((HOPS, B, S_PER, N_OUT), jnp.bfloat16),
            pltpu.VMEM((TILE_S, N_OUT), jnp.bfloat16),
            pltpu.VMEM((TILE_S, N_OUT), jnp.bfloat16),
            pltpu.VMEM((TILE_S, N_OUT), jnp.bfloat16),
            pltpu.VMEM((TILE_S, N_OUT), jnp.float32),
            pltpu.SemaphoreType.DMA((HOPS,)),
            pltpu.SemaphoreType.DMA((HOPS,)),
            pltpu.SemaphoreType.DMA,
            pltpu.SemaphoreType.DMA,
            pltpu.SemaphoreType.DMA,
        ],
    )(partial)


def kernel(O, Wo):
    b, s_full, h, d = O.shape
    Ob = O.reshape(b, N_DEV, S_PER, h * d).astype(jnp.bfloat16)
    Wb = Wo.astype(jnp.bfloat16)
    partial = jnp.einsum(
        "bjsk,kn->jbsn", Ob, Wb, preferred_element_type=jnp.float32
    ).astype(jnp.bfloat16)
    return _ring_reduce_scatter(partial)


# baseline (device time: 2320863 ns/iter reference)
import jax
import jax.numpy as jnp
from jax import lax
from jax.experimental import pallas as pl
from jax.experimental.pallas import tpu as pltpu

N_DEV = 8
B = 2
S_PER = 512
N_OUT = 8192
TILE_S = 256
N_TILES = S_PER // TILE_S
HOPS = N_DEV - 1


def _ring_reduce_scatter(partial):

    def body(p_ref, out_ref, acc_hbm, recv_hbm, va, vb, vc, vf32,
             send_sems, recv_sems, sem_a, sem_b, sem_c):
        my = lax.axis_index("i")
        right = lax.rem(my + 1, N_DEV)

        c0 = lax.rem(my + N_DEV - 1, N_DEV)
        init = pltpu.make_async_copy(p_ref.at[c0], acc_hbm, sem_a)
        init.start()
        init.wait()

        for s in range(HOPS):
            rdma = pltpu.make_async_remote_copy(
                src_ref=acc_hbm,
                dst_ref=recv_hbm.at[s],
                send_sem=send_sems.at[s],
                recv_sem=recv_sems.at[s],
                device_id=(right,),
                device_id_type=pl.DeviceIdType.MESH,
            )
            rdma.start()
            rdma.wait()

            c = lax.rem(my + 2 * N_DEV - 2 - s, N_DEV)
            final = s == HOPS - 1
            for b in range(B):
                for t in range(N_TILES):
                    sl = pl.ds(t * TILE_S, TILE_S)
                    cp_r = pltpu.make_async_copy(recv_hbm.at[s, b, sl], va, sem_a)
                    cp_l = pltpu.make_async_copy(p_ref.at[c, b, sl], vb, sem_b)
                    cp_r.start()
                    cp_l.start()
                    cp_r.wait()
                    cp_l.wait()
                    if final:
                        vf32[...] = (va[...].astype(jnp.float32)
                                     + vb[...].astype(jnp.float32))
                        cp_o = pltpu.make_async_copy(vf32, out_ref.at[b, sl], sem_c)
                    else:
                        vc[...] = va[...] + vb[...]
                        cp_o = pltpu.make_async_copy(vc, acc_hbm.at[b, sl], sem_c)
                    cp_o.start()
                    cp_o.wait()

    hbm = pltpu.MemorySpace.HBM
    out, _, _ = pl.pallas_call(
        body,
        out_shape=(
            jax.ShapeDtypeStruct((B, S_PER, N_OUT), jnp.float32),
            jax.ShapeDtypeStruct((B, S_PER, N_OUT), jnp.bfloat16),
            jax.ShapeDtypeStruct((HOPS, B, S_PER, N_OUT), jnp.bfloat16),
        ),
        in_specs=[pl.BlockSpec(memory_space=hbm)],
        out_specs=(
            pl.BlockSpec(memory_space=hbm),
            pl.BlockSpec(memory_space=hbm),
            pl.BlockSpec(memory_space=hbm),
        ),
        scratch_shapes=[
            pltpu.VMEM((TILE_S, N_OUT), jnp.bfloat16),
            pltpu.VMEM((TILE_S, N_OUT), jnp.bfloat16),
            pltpu.VMEM((TILE_S, N_OUT), jnp.bfloat16),
            pltpu.VMEM((TILE_S, N_OUT), jnp.float32),
            pltpu.SemaphoreType.DMA((HOPS,)),
            pltpu.SemaphoreType.DMA((HOPS,)),
            pltpu.SemaphoreType.DMA,
            pltpu.SemaphoreType.DMA,
            pltpu.SemaphoreType.DMA,
        ],
    )(partial)
    return out


def kernel(O, Wo):
    b, s_full, h, d = O.shape
    Ob = O.reshape(b, N_DEV, S_PER, h * d).astype(jnp.bfloat16)
    Wb = Wo.astype(jnp.bfloat16)
    partial = jnp.einsum(
        "bjsk,kn->jbsn", Ob, Wb, preferred_element_type=jnp.float32
    ).astype(jnp.bfloat16)
    return _ring_reduce_scatter(partial)


# device time: 1897898 ns/iter; 1.2229x vs baseline; 1.2229x over previous
import jax
import jax.numpy as jnp
from jax import lax
from jax.experimental import pallas as pl
from jax.experimental.pallas import tpu as pltpu

N_DEV = 8
B = 2
S_PER = 512
N_OUT = 8192
TILE_S = 256
N_TILES = S_PER // TILE_S
TILE_F = 128
N_FTILES = S_PER // TILE_F
HOPS = N_DEV - 1


def _ring_reduce_scatter(partial):

    def body(p_ref, out_ref, acc_hbm, recv_hbm, vb, va, vc, vf32,
             send_sems, recv_sems, vb_sems, va_sem, st_sem):
        my = lax.axis_index("i")
        right = lax.rem(my + 1, N_DEV)

        def hop_rdma(s, b):
            return pltpu.make_async_remote_copy(
                src_ref=acc_hbm.at[b],
                dst_ref=recv_hbm.at[s, b],
                send_sem=send_sems.at[s * B + b],
                recv_sem=recv_sems.at[s * B + b],
                device_id=(right,),
                device_id_type=pl.DeviceIdType.MESH,
            )

        c0 = lax.rem(my + N_DEV - 1, N_DEV)
        for b in range(B):
            pltpu.make_async_copy(
                p_ref.at[b, c0], acc_hbm.at[b], va_sem).start()
            pltpu.make_async_copy(
                p_ref.at[b, c0], acc_hbm.at[b], va_sem).wait()
            hop_rdma(0, b).start()

        for s in range(HOPS):
            c = lax.rem(my + 2 * N_DEV - 2 - s, N_DEV)
            final = s == HOPS - 1

            for b in range(B):
                for t in range(N_TILES):
                    pltpu.make_async_copy(
                        p_ref.at[b, c, pl.ds(t * TILE_S, TILE_S)],
                        vb.at[b, t], vb_sems.at[b * N_TILES + t],
                    ).start()

            for b in range(B):
                d = hop_rdma(s, b)
                d.wait_recv()
                d.wait_send()
                for t in range(N_TILES):
                    sl = pl.ds(t * TILE_S, TILE_S)
                    pltpu.make_async_copy(
                        recv_hbm.at[s, b, sl], va, va_sem).start()
                    pltpu.make_async_copy(
                        recv_hbm.at[s, b, sl], va, va_sem).wait()
                    pltpu.make_async_copy(
                        p_ref.at[b, c, sl], vb.at[b, t],
                        vb_sems.at[b * N_TILES + t],
                    ).wait()
                    if not final:
                        vc[...] = va[...] + vb[b, t]
                        pltpu.make_async_copy(
                            vc, acc_hbm.at[b, sl], st_sem).start()
                        pltpu.make_async_copy(
                            vc, acc_hbm.at[b, sl], st_sem).wait()
                    else:
                        for f in range(TILE_S // TILE_F):
                            fsl = pl.ds(f * TILE_F, TILE_F)
                            osl = pl.ds(t * TILE_S + f * TILE_F, TILE_F)
                            vf32[...] = (
                                va[fsl].astype(jnp.float32)
                                + vb[b, t, fsl].astype(jnp.float32))
                            pltpu.make_async_copy(
                                vf32, out_ref.at[b, osl], st_sem).start()
                            pltpu.make_async_copy(
                                vf32, out_ref.at[b, osl], st_sem).wait()
                if not final:
                    hop_rdma(s + 1, b).start()

    hbm = pltpu.MemorySpace.HBM
    out, _, _ = pl.pallas_call(
        body,
        out_shape=(
            jax.ShapeDtypeStruct((B, S_PER, N_OUT), jnp.float32),
            jax.ShapeDtypeStruct((B, S_PER, N_OUT), jnp.bfloat16),
            jax.ShapeDtypeStruct((HOPS, B, S_PER, N_OUT), jnp.bfloat16),
        ),
        in_specs=[pl.BlockSpec(memory_space=hbm)],
        out_specs=(
            pl.BlockSpec(memory_space=hbm),
            pl.BlockSpec(memory_space=hbm),
            pl.BlockSpec(memory_space=hbm),
        ),
        scratch_shapes=[
            pltpu.VMEM((B, N_TILES, TILE_S, N_OUT), jnp.bfloat16),
            pltpu.VMEM((TILE_S, N_OUT), jnp.bfloat16),
            pltpu.VMEM((TILE_S, N_OUT), jnp.bfloat16),
            pltpu.VMEM((TILE_F, N_OUT), jnp.float32),
            pltpu.SemaphoreType.DMA((HOPS * B,)),
            pltpu.SemaphoreType.DMA((HOPS * B,)),
            pltpu.SemaphoreType.DMA((B * N_TILES,)),
            pltpu.SemaphoreType.DMA,
            pltpu.SemaphoreType.DMA,
        ],
    )(partial)
    return out


def kernel(O, Wo):
    b, s_full, h, d = O.shape
    Ob = O.reshape(b, s_full, h * d).astype(jnp.bfloat16)
    Wb = Wo.astype(jnp.bfloat16)
    partial = jnp.matmul(
        Ob, Wb, preferred_element_type=jnp.float32
    ).astype(jnp.bfloat16).reshape(b, N_DEV, S_PER, N_OUT)
    return _ring_reduce_scatter(partial)


# device time: 1479454 ns/iter; 1.5687x vs baseline; 1.2828x over previous
import jax
import jax.numpy as jnp
from jax import lax
from jax.experimental import pallas as pl
from jax.experimental.pallas import tpu as pltpu

N_DEV = 8
B = 2
S_PER = 512
N_OUT = 8192
TILE_S = 256
N_TILES = S_PER // TILE_S
TILE_F = 128
N_FTILES = S_PER // TILE_F
HOPS = N_DEV - 1


def _ring_reduce_scatter(partial):

    def body(p_ref, out_ref, recv_hbm, acc_v, vb, va, vf32,
             send_sems, recv_sems, vb_sems, va_sem, st_sem):
        my = lax.axis_index("i")
        right = lax.rem(my + 1, N_DEV)

        def hop_rdma(s, b):
            return pltpu.make_async_remote_copy(
                src_ref=acc_v.at[b],
                dst_ref=recv_hbm.at[s, b],
                send_sem=send_sems.at[s * B + b],
                recv_sem=recv_sems.at[s * B + b],
                device_id=(right,),
                device_id_type=pl.DeviceIdType.MESH,
            )

        c0 = lax.rem(my + N_DEV - 1, N_DEV)
        for b in range(B):
            pltpu.make_async_copy(
                p_ref.at[b, c0], acc_v.at[b], va_sem).start()
            pltpu.make_async_copy(
                p_ref.at[b, c0], acc_v.at[b], va_sem).wait()
            hop_rdma(0, b).start()

        for s in range(HOPS):
            c = lax.rem(my + 2 * N_DEV - 2 - s, N_DEV)
            final = s == HOPS - 1

            for b in range(B):
                for t in range(N_TILES):
                    pltpu.make_async_copy(
                        p_ref.at[b, c, pl.ds(t * TILE_S, TILE_S)],
                        vb.at[b, t], vb_sems.at[b * N_TILES + t],
                    ).start()

            for b in range(B):
                d = hop_rdma(s, b)
                d.wait_recv()
                d.wait_send()
                for t in range(N_TILES):
                    sl = pl.ds(t * TILE_S, TILE_S)
                    pltpu.make_async_copy(
                        recv_hbm.at[s, b, sl], va, va_sem).start()
                    pltpu.make_async_copy(
                        recv_hbm.at[s, b, sl], va, va_sem).wait()
                    pltpu.make_async_copy(
                        p_ref.at[b, c, sl], vb.at[b, t],
                        vb_sems.at[b * N_TILES + t],
                    ).wait()
                    if not final:
                        acc_v[b, sl] = va[...] + vb[b, t]
                    else:
                        for f in range(TILE_S // TILE_F):
                            fsl = pl.ds(f * TILE_F, TILE_F)
                            osl = pl.ds(t * TILE_S + f * TILE_F, TILE_F)
                            vf32[...] = (
                                va[fsl].astype(jnp.float32)
                                + vb[b, t, fsl].astype(jnp.float32))
                            pltpu.make_async_copy(
                                vf32, out_ref.at[b, osl], st_sem).start()
                            pltpu.make_async_copy(
                                vf32, out_ref.at[b, osl], st_sem).wait()
                if not final:
                    hop_rdma(s + 1, b).start()

    hbm = pltpu.MemorySpace.HBM
    out, _ = pl.pallas_call(
        body,
        out_shape=(
            jax.ShapeDtypeStruct((B, S_PER, N_OUT), jnp.float32),
            jax.ShapeDtypeStruct((HOPS, B, S_PER, N_OUT), jnp.bfloat16),
        ),
        in_specs=[pl.BlockSpec(memory_space=hbm)],
        out_specs=(
            pl.BlockSpec(memory_space=hbm),
            pl.BlockSpec(memory_space=hbm),
        ),
        scratch_shapes=[
            pltpu.VMEM((B, S_PER, N_OUT), jnp.bfloat16),
            pltpu.VMEM((B, N_TILES, TILE_S, N_OUT), jnp.bfloat16),
            pltpu.VMEM((TILE_S, N_OUT), jnp.bfloat16),
            pltpu.VMEM((TILE_F, N_OUT), jnp.float32),
            pltpu.SemaphoreType.DMA((HOPS * B,)),
            pltpu.SemaphoreType.DMA((HOPS * B,)),
            pltpu.SemaphoreType.DMA((B * N_TILES,)),
            pltpu.SemaphoreType.DMA,
            pltpu.SemaphoreType.DMA,
        ],
        compiler_params=pltpu.CompilerParams(
            vmem_limit_bytes=48 * 1024 * 1024,
        ),
    )(partial)
    return out


def kernel(O, Wo):
    b, s_full, h, d = O.shape
    Ob = O.reshape(b, s_full, h * d).astype(jnp.bfloat16)
    Wb = Wo.astype(jnp.bfloat16)
    partial = jnp.matmul(
        Ob, Wb, preferred_element_type=jnp.float32
    ).astype(jnp.bfloat16).reshape(b, N_DEV, S_PER, N_OUT)
    return _ring_reduce_scatter(partial)


# device time: 1388495 ns/iter; 1.6715x vs baseline; 1.0655x over previous
import jax
import jax.numpy as jnp
from jax import lax
from jax.experimental import pallas as pl
from jax.experimental.pallas import tpu as pltpu

N_DEV = 8
B = 2
S_PER = 512
N_OUT = 8192
K = 1024
TILE_M = 128
N_MTILES = S_PER // TILE_M
HOPS = N_DEV - 1


def _fused_matmul_reduce_scatter(Ob, Wb):

    def body(o_ref, w_ref, out_ref, recv_hbm, w_v, acc_v, vb, o_v, va, vf32,
             send_sems, recv_sems, o_sems, va_sem, st_sem):
        my = lax.axis_index("i")
        right = lax.rem(my + 1, N_DEV)

        def hop_rdma(s, b):
            return pltpu.make_async_remote_copy(
                src_ref=acc_v.at[b],
                dst_ref=recv_hbm.at[s, b],
                send_sem=send_sems.at[s * B + b],
                recv_sem=recv_sems.at[s * B + b],
                device_id=(right,),
                device_id_type=pl.DeviceIdType.MESH,
            )

        def load_o(c):
            for b in range(B):
                pltpu.make_async_copy(
                    o_ref.at[b, c], o_v.at[b], o_sems.at[b]).start()

        def compute_chunk(c, dst):
            for b in range(B):
                pltpu.make_async_copy(
                    o_ref.at[b, c], o_v.at[b], o_sems.at[b]).wait()

                def dot_tile(r, _):
                    sl = pl.ds(r * TILE_M, TILE_M)
                    dst[b, sl] = jnp.dot(
                        o_v[b, sl], w_v[...],
                        preferred_element_type=jnp.float32,
                    ).astype(jnp.bfloat16)
                    return 0

                lax.fori_loop(0, N_MTILES, dot_tile, 0)

        pltpu.make_async_copy(w_ref, w_v, va_sem).start()
        c0 = lax.rem(my + N_DEV - 1, N_DEV)
        load_o(c0)
        pltpu.make_async_copy(w_ref, w_v, va_sem).wait()
        compute_chunk(c0, acc_v)
        for b in range(B):
            hop_rdma(0, b).start()

        def hop(s, _):
            c = lax.rem(my + 2 * N_DEV - 2 - s, N_DEV)
            final = s == HOPS - 1
            load_o(c)
            compute_chunk(c, vb)

            for b in range(B):
                d = hop_rdma(s, b)
                d.wait_recv()
                d.wait_send()

                def add_tile(t, _):
                    sl = pl.ds(t * TILE_M, TILE_M)
                    pltpu.make_async_copy(
                        recv_hbm.at[s, b, sl], va, va_sem).start()
                    pltpu.make_async_copy(
                        recv_hbm.at[s, b, sl], va, va_sem).wait()

                    @pl.when(jnp.logical_not(final))
                    def _():
                        acc_v[b, sl] = va[...] + vb[b, sl]

                    @pl.when(final)
                    def _():
                        vf32[...] = (va[...].astype(jnp.float32)
                                     + vb[b, sl].astype(jnp.float32))
                        pltpu.make_async_copy(
                            vf32, out_ref.at[b, sl], st_sem).start()
                        pltpu.make_async_copy(
                            vf32, out_ref.at[b, sl], st_sem).wait()

                    return 0

                lax.fori_loop(0, N_MTILES, add_tile, 0)

                @pl.when(jnp.logical_not(final))
                def _():
                    hop_rdma(s + 1, b).start()
            return 0

        lax.fori_loop(0, HOPS, hop, 0)

    hbm = pltpu.MemorySpace.HBM
    out, _ = pl.pallas_call(
        body,
        out_shape=(
            jax.ShapeDtypeStruct((B, S_PER, N_OUT), jnp.float32),
            jax.ShapeDtypeStruct((HOPS, B, S_PER, N_OUT), jnp.bfloat16),
        ),
        in_specs=[
            pl.BlockSpec(memory_space=hbm),
            pl.BlockSpec(memory_space=hbm),
        ],
        out_specs=(
            pl.BlockSpec(memory_space=hbm),
            pl.BlockSpec(memory_space=hbm),
        ),
        scratch_shapes=[
            pltpu.VMEM((K, N_OUT), jnp.bfloat16),
            pltpu.VMEM((B, S_PER, N_OUT), jnp.bfloat16),
            pltpu.VMEM((B, S_PER, N_OUT), jnp.bfloat16),
            pltpu.VMEM((B, S_PER, K), jnp.bfloat16),
            pltpu.VMEM((TILE_M, N_OUT), jnp.bfloat16),
            pltpu.VMEM((TILE_M, N_OUT), jnp.float32),
            pltpu.SemaphoreType.DMA((HOPS * B,)),
            pltpu.SemaphoreType.DMA((HOPS * B,)),
            pltpu.SemaphoreType.DMA((B,)),
            pltpu.SemaphoreType.DMA,
            pltpu.SemaphoreType.DMA,
        ],
        compiler_params=pltpu.CompilerParams(
            vmem_limit_bytes=62 * 1024 * 1024,
        ),
    )(Ob, Wb)
    return out


def kernel(O, Wo):
    b, s_full, h, d = O.shape
    Ob = O.reshape(b, N_DEV, S_PER, h * d).astype(jnp.bfloat16)
    Wb = Wo.astype(jnp.bfloat16)
    return _fused_matmul_reduce_scatter(Ob, Wb)
